# baseline (speedup 1.0000x reference)
import jax
import jax.numpy as jnp
from jax.experimental import pallas as pl
from jax.experimental.pallas import tpu as pltpu

TOPK = 8
DOC_LEN = 100
N_KEYS = 100000
D = 128
Q = 64

BLK = 4096
SUB = BLK // 128
NGRP = SUB // 8
NBLK = (N_KEYS + BLK - 1) // BLK

_NEG_INF = float("-inf")
_BIG_I32 = 2**30

_SORT8 = [
    (0, 1), (2, 3), (4, 5), (6, 7),
    (0, 2), (1, 3), (4, 6), (5, 7),
    (1, 2), (5, 6),
    (0, 4), (1, 5), (2, 6), (3, 7),
    (2, 4), (3, 5),
    (1, 2), (3, 4), (5, 6),
]
_BMERGE8 = [
    (0, 4), (1, 5), (2, 6), (3, 7),
    (0, 2), (1, 3), (4, 6), (5, 7),
    (0, 1), (2, 3), (4, 5), (6, 7),
]


def _ce(v, x, a, b):
    c = v[a] >= v[b]
    va, vb = jnp.where(c, v[a], v[b]), jnp.where(c, v[b], v[a])
    xa, xb = jnp.where(c, x[a], x[b]), jnp.where(c, x[b], x[a])
    v[a], v[b], x[a], x[b] = va, vb, xa, xb


def _sort8(v, x):
    for a, b in _SORT8:
        _ce(v, x, a, b)


def _top8_merge(av, ax, bv, bx):
    mv, mx = [], []
    for j in range(TOPK):
        c = av[j] >= bv[TOPK - 1 - j]
        mv.append(jnp.where(c, av[j], bv[TOPK - 1 - j]))
        mx.append(jnp.where(c, ax[j], bx[TOPK - 1 - j]))
    for a, b in _BMERGE8:
        c = mv[a] >= mv[b]
        mv[a], mv[b] = jnp.where(c, mv[a], mv[b]), jnp.where(c, mv[b], mv[a])
        mx[a], mx[b] = jnp.where(c, mx[a], mx[b]), jnp.where(c, mx[b], mx[a])
    return mv, mx


def _topk_kernel(start_ref, q_ref, e_ref, vals_out, idx_out, rv_ref, ri_ref):
    i = pl.program_id(0)

    @pl.when(i == 0)
    def _init():
        rv_ref[...] = jnp.full((Q, TOPK * 128), _NEG_INF, jnp.float32)
        ri_ref[...] = jnp.zeros((Q, TOPK * 128), jnp.int32)

    q = q_ref[...]
    denom = jnp.clip(jnp.sum(jnp.abs(q), axis=1, keepdims=True), 1e-12, None)
    qn = q / denom

    s = jax.lax.dot_general(
        qn, e_ref[...], (((1,), (1,)), ((), ())),
        preferred_element_type=jnp.float32,
    )

    start = start_ref[0]
    lane = jax.lax.broadcasted_iota(jnp.int32, (Q, 128), 1)
    base = i * BLK

    groups = []
    for g in range(NGRP):
        gv, gx = [], []
        for c in range(8):
            off = g * 1024 + c * 128
            col = lane + (base + off)
            x = s[:, off:off + 128]
            in_doc = (col - start).astype(jnp.uint32) < DOC_LEN
            oob = lane >= (N_KEYS - base - off)
            gv.append(jnp.where(in_doc | oob, _NEG_INF, x))
            gx.append(col)
        _sort8(gv, gx)
        groups.append((gv, gx))
    while len(groups) > 1:
        groups = [
            _top8_merge(groups[k][0], groups[k][1],
                        groups[k + 1][0], groups[k + 1][1])
            for k in range(0, len(groups), 2)
        ]
    nv, nx = groups[0]

    cv = [rv_ref[:, j * 128:(j + 1) * 128] for j in range(TOPK)]
    cx = [ri_ref[:, j * 128:(j + 1) * 128] for j in range(TOPK)]
    mv, mx = _top8_merge(cv, cx, nv, nx)
    rv_ref[...] = jnp.concatenate(mv, axis=1)
    ri_ref[...] = jnp.concatenate(mx, axis=1)

    @pl.when(i == NBLK - 1)
    def _emit():
        v_all = jnp.concatenate(mv, axis=1)
        i_all = jnp.concatenate(mx, axis=1)
        ov, oi = [], []
        for _ in range(TOPK):
            m = jnp.max(v_all, axis=1, keepdims=True)
            am = jnp.min(jnp.where(v_all == m, i_all, _BIG_I32),
                         axis=1, keepdims=True)
            ov.append(m)
            oi.append(am)
            v_all = jnp.where((v_all == m) & (i_all == am), _NEG_INF, v_all)
        vals_out[...] = jnp.concatenate(ov, axis=1).T
        idx_out[...] = jnp.concatenate(oi, axis=1).T


def kernel(query, embeddings, doc_id):
    corpus = embeddings.T
    start = (jnp.asarray(doc_id, jnp.int32) * DOC_LEN).reshape((1,))
    grid_spec = pltpu.PrefetchScalarGridSpec(
        num_scalar_prefetch=1,
        grid=(NBLK,),
        in_specs=[
            pl.BlockSpec((Q, D), lambda i, s: (0, 0)),
            pl.BlockSpec((BLK, D), lambda i, s: (i, 0)),
        ],
        out_specs=[
            pl.BlockSpec((TOPK, Q), lambda i, s: (0, 0)),
            pl.BlockSpec((TOPK, Q), lambda i, s: (0, 0)),
        ],
        scratch_shapes=[
            pltpu.VMEM((Q, TOPK * 128), jnp.float32),
            pltpu.VMEM((Q, TOPK * 128), jnp.int32),
        ],
    )
    values, indices = pl.pallas_call(
        _topk_kernel,
        grid_spec=grid_spec,
        out_shape=[
            jax.ShapeDtypeStruct((TOPK, Q), jnp.float32),
            jax.ShapeDtypeStruct((TOPK, Q), jnp.int32),
        ],
    )(start, query, corpus)
    return values.T, indices.T


# speedup vs baseline: 1.1312x; 1.1312x over previous
import jax
import jax.numpy as jnp
from jax.experimental import pallas as pl
from jax.experimental.pallas import tpu as pltpu

TOPK = 8
DOC_LEN = 100
N_KEYS = 100000
D = 128
Q = 64

BLK = 8192
SUB = BLK // 128
NGRP = SUB // 8
NBLK = (N_KEYS + BLK - 1) // BLK

_NEG_INF = float("-inf")
_BIG_I32 = 2**30

_SORT8 = [
    (0, 1), (2, 3), (4, 5), (6, 7),
    (0, 2), (1, 3), (4, 6), (5, 7),
    (1, 2), (5, 6),
    (0, 4), (1, 5), (2, 6), (3, 7),
    (2, 4), (3, 5),
    (1, 2), (3, 4), (5, 6),
]
_BMERGE8 = [
    (0, 4), (1, 5), (2, 6), (3, 7),
    (0, 2), (1, 3), (4, 6), (5, 7),
    (0, 1), (2, 3), (4, 5), (6, 7),
]


def _ce(v, x, a, b):
    c = v[a] >= v[b]
    va, vb = jnp.maximum(v[a], v[b]), jnp.minimum(v[a], v[b])
    xa, xb = jnp.where(c, x[a], x[b]), jnp.where(c, x[b], x[a])
    v[a], v[b], x[a], x[b] = va, vb, xa, xb


def _sort8(v, x):
    for a, b in _SORT8:
        _ce(v, x, a, b)


def _top8_merge(av, ax, bv, bx):
    mv, mx = [], []
    for j in range(TOPK):
        c = av[j] >= bv[TOPK - 1 - j]
        mv.append(jnp.maximum(av[j], bv[TOPK - 1 - j]))
        mx.append(jnp.where(c, ax[j], bx[TOPK - 1 - j]))
    for a, b in _BMERGE8:
        _ce(mv, mx, a, b)
    return mv, mx


def _topk_kernel(start_ref, q_ref, e_ref, vals_out, idx_out, rv_ref, ri_ref):
    i = pl.program_id(0)

    @pl.when(i == 0)
    def _init():
        rv_ref[...] = jnp.full((Q, TOPK * 128), _NEG_INF, jnp.float32)
        ri_ref[...] = jnp.zeros((Q, TOPK * 128), jnp.int32)

    q = q_ref[...]
    denom = jnp.clip(jnp.sum(jnp.abs(q), axis=1, keepdims=True), 1e-12, None)
    qn = q / denom

    s = jax.lax.dot_general(
        qn, e_ref[...], (((1,), (1,)), ((), ())),
        preferred_element_type=jnp.float32,
    )

    start = start_ref[0]
    lane = jax.lax.broadcasted_iota(jnp.int32, (Q, 128), 1)
    base = i * BLK

    groups = []
    for g in range(NGRP):
        gv, gx = [], []
        for c in range(8):
            off = g * 1024 + c * 128
            x = s[:, off:off + 128]
            in_doc = (lane + (base + off - start)).astype(jnp.uint32) < DOC_LEN
            oob = lane >= (N_KEYS - base - off)
            gv.append(jnp.where(in_doc | oob, _NEG_INF, x))
            gx.append(jnp.full((Q, 128), base + off, jnp.int32))
        _sort8(gv, gx)
        groups.append((gv, gx))
    while len(groups) > 1:
        groups = [
            _top8_merge(groups[k][0], groups[k][1],
                        groups[k + 1][0], groups[k + 1][1])
            for k in range(0, len(groups), 2)
        ]
    nv, nx = groups[0]

    cv = [rv_ref[:, j * 128:(j + 1) * 128] for j in range(TOPK)]
    cx = [ri_ref[:, j * 128:(j + 1) * 128] for j in range(TOPK)]
    mv, mx = _top8_merge(cv, cx, nv, nx)
    rv_ref[...] = jnp.concatenate(mv, axis=1)
    ri_ref[...] = jnp.concatenate(mx, axis=1)

    @pl.when(i == NBLK - 1)
    def _emit():
        v_all = jnp.concatenate(mv, axis=1)
        lane_all = jax.lax.broadcasted_iota(jnp.int32, (Q, TOPK * 128), 1) & 127
        i_all = jnp.concatenate(mx, axis=1) + lane_all
        ov, oi = [], []
        for _ in range(TOPK):
            m = jnp.max(v_all, axis=1, keepdims=True)
            am = jnp.min(jnp.where(v_all == m, i_all, _BIG_I32),
                         axis=1, keepdims=True)
            ov.append(m)
            oi.append(am)
            v_all = jnp.where((v_all == m) & (i_all == am), _NEG_INF, v_all)
        vals_out[...] = jnp.concatenate(ov, axis=1).T
        idx_out[...] = jnp.concatenate(oi, axis=1).T


def kernel(query, embeddings, doc_id):
    corpus = embeddings.T
    start = (jnp.asarray(doc_id, jnp.int32) * DOC_LEN).reshape((1,))
    grid_spec = pltpu.PrefetchScalarGridSpec(
        num_scalar_prefetch=1,
        grid=(NBLK,),
        in_specs=[
            pl.BlockSpec((Q, D), lambda i, s: (0, 0)),
            pl.BlockSpec((BLK, D), lambda i, s: (i, 0)),
        ],
        out_specs=[
            pl.BlockSpec((TOPK, Q), lambda i, s: (0, 0)),
            pl.BlockSpec((TOPK, Q), lambda i, s: (0, 0)),
        ],
        scratch_shapes=[
            pltpu.VMEM((Q, TOPK * 128), jnp.float32),
            pltpu.VMEM((Q, TOPK * 128), jnp.int32),
        ],
    )
    values, indices = pl.pallas_call(
        _topk_kernel,
        grid_spec=grid_spec,
        out_shape=[
            jax.ShapeDtypeStruct((TOPK, Q), jnp.float32),
            jax.ShapeDtypeStruct((TOPK, Q), jnp.int32),
        ],
    )(start, query, corpus)
    return values.T, indices.T


# speedup vs baseline: 1.1368x; 1.0049x over previous
import jax
import jax.numpy as jnp
from jax.experimental import pallas as pl
from jax.experimental.pallas import tpu as pltpu

TOPK = 8
DOC_LEN = 100
N_KEYS = 100000
D = 128
Q = 64

BLK = 8192
SUB = BLK // 128
NGRP = SUB // 8
NBLK = (N_KEYS + BLK - 1) // BLK

_NEG_INF = float("-inf")
_BIG_I32 = 2**30

_SORT8 = [
    (0, 1), (2, 3), (4, 5), (6, 7),
    (0, 2), (1, 3), (4, 6), (5, 7),
    (1, 2), (5, 6),
    (0, 4), (1, 5), (2, 6), (3, 7),
    (2, 4), (3, 5),
    (1, 2), (3, 4), (5, 6),
]
_BMERGE8 = [
    (0, 4), (1, 5), (2, 6), (3, 7),
    (0, 2), (1, 3), (4, 6), (5, 7),
    (0, 1), (2, 3), (4, 5), (6, 7),
]


def _ce(v, x, a, b):
    c = v[a] >= v[b]
    va, vb = jnp.maximum(v[a], v[b]), jnp.minimum(v[a], v[b])
    xa, xb = jnp.where(c, x[a], x[b]), jnp.where(c, x[b], x[a])
    v[a], v[b], x[a], x[b] = va, vb, xa, xb


def _sort8(v, x):
    for a, b in _SORT8:
        _ce(v, x, a, b)


def _top8_merge(av, ax, bv, bx):
    mv, mx = [], []
    for j in range(TOPK):
        c = av[j] >= bv[TOPK - 1 - j]
        mv.append(jnp.maximum(av[j], bv[TOPK - 1 - j]))
        mx.append(jnp.where(c, ax[j], bx[TOPK - 1 - j]))
    for a, b in _BMERGE8:
        _ce(mv, mx, a, b)
    return mv, mx


def _topk_kernel(start_ref, q_ref, e_ref, vals_out, idx_out, rv_ref, ri_ref):
    i = pl.program_id(0)

    @pl.when(i == 0)
    def _init():
        rv_ref[...] = jnp.full((Q, TOPK * 128), _NEG_INF, jnp.float32)
        ri_ref[...] = jnp.zeros((Q, TOPK * 128), jnp.int32)

    q = q_ref[...]
    denom = jnp.clip(jnp.sum(jnp.abs(q), axis=1, keepdims=True), 1e-12, None)
    qn = q / denom

    s = jax.lax.dot_general(
        qn, e_ref[...], (((1,), (1,)), ((), ())),
        preferred_element_type=jnp.float32,
    )

    start = start_ref[0]
    lane = jax.lax.broadcasted_iota(jnp.int32, (Q, 128), 1)
    base = i * BLK

    groups = []
    for g in range(NGRP):
        gv, gx = [], []
        for c in range(8):
            off = g * 1024 + c * 128
            x = s[:, off:off + 128]
            in_doc = (lane + (base + off - start)).astype(jnp.uint32) < DOC_LEN
            oob = lane >= (N_KEYS - base - off)
            gv.append(jnp.where(in_doc | oob, _NEG_INF, x))
            gx.append(jnp.full((Q, 128), base + off, jnp.int32))
        _sort8(gv, gx)
        groups.append((gv, gx))
    while len(groups) > 1:
        groups = [
            _top8_merge(groups[k][0], groups[k][1],
                        groups[k + 1][0], groups[k + 1][1])
            for k in range(0, len(groups), 2)
        ]
    nv, nx = groups[0]

    cv = [rv_ref[:, j * 128:(j + 1) * 128] for j in range(TOPK)]
    cx = [ri_ref[:, j * 128:(j + 1) * 128] for j in range(TOPK)]
    mv, mx = _top8_merge(cv, cx, nv, nx)
    rv_ref[...] = jnp.concatenate(mv, axis=1)
    ri_ref[...] = jnp.concatenate(mx, axis=1)

    @pl.when(i == NBLK - 1)
    def _emit():
        fv = list(mv)
        fx = [mx[j] + lane for j in range(TOPK)]
        ov, oi = [], []
        for _ in range(TOPK):
            m = jnp.max(fv[0], axis=1, keepdims=True)
            am = jnp.min(jnp.where(fv[0] == m, fx[0], _BIG_I32),
                         axis=1, keepdims=True)
            ov.append(m)
            oi.append(am)
            hit = (fv[0] == m) & (fx[0] == am)
            for j in range(TOPK - 1):
                fv[j] = jnp.where(hit, fv[j + 1], fv[j])
                fx[j] = jnp.where(hit, fx[j + 1], fx[j])
            fv[TOPK - 1] = jnp.where(hit, _NEG_INF, fv[TOPK - 1])
        vals_out[...] = jnp.concatenate(ov, axis=1).T
        idx_out[...] = jnp.concatenate(oi, axis=1).T


def kernel(query, embeddings, doc_id):
    corpus = embeddings.T
    start = (jnp.asarray(doc_id, jnp.int32) * DOC_LEN).reshape((1,))
    grid_spec = pltpu.PrefetchScalarGridSpec(
        num_scalar_prefetch=1,
        grid=(NBLK,),
        in_specs=[
            pl.BlockSpec((Q, D), lambda i, s: (0, 0)),
            pl.BlockSpec((BLK, D), lambda i, s: (i, 0)),
        ],
        out_specs=[
            pl.BlockSpec((TOPK, Q), lambda i, s: (0, 0)),
            pl.BlockSpec((TOPK, Q), lambda i, s: (0, 0)),
        ],
        scratch_shapes=[
            pltpu.VMEM((Q, TOPK * 128), jnp.float32),
            pltpu.VMEM((Q, TOPK * 128), jnp.int32),
        ],
    )
    values, indices = pl.pallas_call(
        _topk_kernel,
        grid_spec=grid_spec,
        out_shape=[
            jax.ShapeDtypeStruct((TOPK, Q), jnp.float32),
            jax.ShapeDtypeStruct((TOPK, Q), jnp.int32),
        ],
    )(start, query, corpus)
    return values.T, indices.T


# speedup vs baseline: 1.1730x; 1.0318x over previous
import jax
import jax.numpy as jnp
from jax.experimental import pallas as pl
from jax.experimental.pallas import tpu as pltpu

TOPK = 8
DOC_LEN = 100
N_KEYS = 100000
D = 128
Q = 64

BLK = 8192
SUB = BLK // 128
NGRP = SUB // 8
NBLK = N_KEYS // BLK
TAIL_BLK = 2048
TAIL_IDX = N_KEYS // BLK * (BLK // TAIL_BLK)

_NEG_INF = float("-inf")
_BIG_I32 = 2**30

_SORT8 = [
    (0, 1), (2, 3), (4, 5), (6, 7),
    (0, 2), (1, 3), (4, 6), (5, 7),
    (1, 2), (5, 6),
    (0, 4), (1, 5), (2, 6), (3, 7),
    (2, 4), (3, 5),
    (1, 2), (3, 4), (5, 6),
]
_BMERGE8 = [
    (0, 4), (1, 5), (2, 6), (3, 7),
    (0, 2), (1, 3), (4, 6), (5, 7),
    (0, 1), (2, 3), (4, 5), (6, 7),
]


def _ce(v, x, a, b):
    c = v[a] >= v[b]
    va, vb = jnp.maximum(v[a], v[b]), jnp.minimum(v[a], v[b])
    xa, xb = jnp.where(c, x[a], x[b]), jnp.where(c, x[b], x[a])
    v[a], v[b], x[a], x[b] = va, vb, xa, xb


def _sort8(v, x):
    for a, b in _SORT8:
        _ce(v, x, a, b)


def _top8_merge(av, ax, bv, bx):
    mv, mx = [], []
    for j in range(TOPK):
        c = av[j] >= bv[TOPK - 1 - j]
        mv.append(jnp.maximum(av[j], bv[TOPK - 1 - j]))
        mx.append(jnp.where(c, ax[j], bx[TOPK - 1 - j]))
    for a, b in _BMERGE8:
        _ce(mv, mx, a, b)
    return mv, mx


def _sort_block(s, lane, base, start, width, masked_tail):
    groups = []
    for g in range(width // 1024):
        gv, gx = [], []
        for c in range(8):
            off = g * 1024 + c * 128
            x = s[:, off:off + 128]
            bad = (lane + (base + off - start)).astype(jnp.uint32) < DOC_LEN
            if masked_tail:
                bad = bad | (lane >= (N_KEYS - base - off))
            gv.append(jnp.where(bad, _NEG_INF, x))
            gx.append(jnp.full((Q, 128), base + off, jnp.int32))
        _sort8(gv, gx)
        groups.append((gv, gx))
    while len(groups) > 1:
        groups = [
            _top8_merge(groups[k][0], groups[k][1],
                        groups[k + 1][0], groups[k + 1][1])
            for k in range(0, len(groups), 2)
        ]
    return groups[0]


def _topk_kernel(start_ref, q_ref, e_ref, t_ref, vals_out, idx_out,
                 rv_ref, ri_ref):
    i = pl.program_id(0)
    start = start_ref[0]
    lane = jax.lax.broadcasted_iota(jnp.int32, (Q, 128), 1)

    q = q_ref[...]
    denom = jnp.clip(jnp.sum(jnp.abs(q), axis=1, keepdims=True), 1e-12, None)
    qn = q / denom

    @pl.when(i == 0)
    def _init():
        ts = jax.lax.dot_general(
            qn, t_ref[...], (((1,), (1,)), ((), ())),
            preferred_element_type=jnp.float32,
        )
        tv, tx = _sort_block(ts, lane, NBLK * BLK, start, TAIL_BLK,
                             masked_tail=True)
        rv_ref[...] = jnp.concatenate(tv, axis=1)
        ri_ref[...] = jnp.concatenate(tx, axis=1)

    s = jax.lax.dot_general(
        qn, e_ref[...], (((1,), (1,)), ((), ())),
        preferred_element_type=jnp.float32,
    )

    nv, nx = _sort_block(s, lane, i * BLK, start, BLK, masked_tail=False)

    cv = [rv_ref[:, j * 128:(j + 1) * 128] for j in range(TOPK)]
    cx = [ri_ref[:, j * 128:(j + 1) * 128] for j in range(TOPK)]
    mv, mx = _top8_merge(cv, cx, nv, nx)
    rv_ref[...] = jnp.concatenate(mv, axis=1)
    ri_ref[...] = jnp.concatenate(mx, axis=1)

    @pl.when(i == NBLK - 1)
    def _emit():
        fv = list(mv)
        fx = [mx[j] + lane for j in range(TOPK)]
        ov, oi = [], []
        for _ in range(TOPK):
            m = jnp.max(fv[0], axis=1, keepdims=True)
            am = jnp.min(jnp.where(fv[0] == m, fx[0], _BIG_I32),
                         axis=1, keepdims=True)
            ov.append(m)
            oi.append(am)
            hit = (fv[0] == m) & (fx[0] == am)
            for j in range(TOPK - 1):
                fv[j] = jnp.where(hit, fv[j + 1], fv[j])
                fx[j] = jnp.where(hit, fx[j + 1], fx[j])
            fv[TOPK - 1] = jnp.where(hit, _NEG_INF, fv[TOPK - 1])
        vals_out[...] = jnp.concatenate(ov, axis=1).T
        idx_out[...] = jnp.concatenate(oi, axis=1).T


def kernel(query, embeddings, doc_id):
    corpus = embeddings.T
    start = (jnp.asarray(doc_id, jnp.int32) * DOC_LEN).reshape((1,))
    grid_spec = pltpu.PrefetchScalarGridSpec(
        num_scalar_prefetch=1,
        grid=(NBLK,),
        in_specs=[
            pl.BlockSpec((Q, D), lambda i, s: (0, 0)),
            pl.BlockSpec((BLK, D), lambda i, s: (i, 0)),
            pl.BlockSpec((TAIL_BLK, D), lambda i, s: (TAIL_IDX, 0)),
        ],
        out_specs=[
            pl.BlockSpec((TOPK, Q), lambda i, s: (0, 0)),
            pl.BlockSpec((TOPK, Q), lambda i, s: (0, 0)),
        ],
        scratch_shapes=[
            pltpu.VMEM((Q, TOPK * 128), jnp.float32),
            pltpu.VMEM((Q, TOPK * 128), jnp.int32),
        ],
    )
    values, indices = pl.pallas_call(
        _topk_kernel,
        grid_spec=grid_spec,
        out_shape=[
            jax.ShapeDtypeStruct((TOPK, Q), jnp.float32),
            jax.ShapeDtypeStruct((TOPK, Q), jnp.int32),
        ],
    )(start, query, corpus, corpus)
    return values.T, indices.T


# speedup vs baseline: 1.1787x; 1.0049x over previous
import jax
import jax.numpy as jnp
from jax.experimental import pallas as pl
from jax.experimental.pallas import tpu as pltpu

TOPK = 8
DOC_LEN = 100
N_KEYS = 100000
D = 128
Q = 64

BLK = 8192
SUB = BLK // 128
NGRP = SUB // 8
NBLK = N_KEYS // BLK
TAIL_BLK = 2048
TAIL_IDX = N_KEYS // BLK * (BLK // TAIL_BLK)

_NEG_INF = float("-inf")
_BIG_I32 = 2**30

_SORT8 = [
    (0, 1), (2, 3), (4, 5), (6, 7),
    (0, 2), (1, 3), (4, 6), (5, 7),
    (1, 2), (5, 6),
    (0, 4), (1, 5), (2, 6), (3, 7),
    (2, 4), (3, 5),
    (1, 2), (3, 4), (5, 6),
]
_BMERGE8 = [
    (0, 4), (1, 5), (2, 6), (3, 7),
    (0, 2), (1, 3), (4, 6), (5, 7),
    (0, 1), (2, 3), (4, 5), (6, 7),
]


def _ce(v, x, a, b):
    c = v[a] >= v[b]
    va, vb = jnp.maximum(v[a], v[b]), jnp.minimum(v[a], v[b])
    xa, xb = jnp.where(c, x[a], x[b]), jnp.where(c, x[b], x[a])
    v[a], v[b], x[a], x[b] = va, vb, xa, xb


def _sort8(v, x):
    for a, b in _SORT8:
        _ce(v, x, a, b)


def _top8_merge(av, ax, bv, bx):
    mv, mx = [], []
    for j in range(TOPK):
        c = av[j] >= bv[TOPK - 1 - j]
        mv.append(jnp.maximum(av[j], bv[TOPK - 1 - j]))
        mx.append(jnp.where(c, ax[j], bx[TOPK - 1 - j]))
    for a, b in _BMERGE8:
        _ce(mv, mx, a, b)
    return mv, mx


def _sort_block(s, lane, base, start, width, masked_tail):
    groups = []
    for g in range(width // 1024):
        gv, gx = [], []
        for c in range(8):
            off = g * 1024 + c * 128
            x = s[:, off:off + 128]
            bad = (lane + (base + off - start)).astype(jnp.uint32) < DOC_LEN
            if masked_tail:
                bad = bad | (lane >= (N_KEYS - base - off))
            gv.append(jnp.where(bad, _NEG_INF, x))
            gx.append(jnp.full((Q, 128), base + off, jnp.int32))
        _sort8(gv, gx)
        groups.append((gv, gx))
    while len(groups) > 1:
        groups = [
            _top8_merge(groups[k][0], groups[k][1],
                        groups[k + 1][0], groups[k + 1][1])
            for k in range(0, len(groups), 2)
        ]
    return groups[0]


def _topk_kernel(start_ref, q_ref, e_ref, t_ref, vals_out, idx_out,
                 rv_ref, ri_ref):
    i = pl.program_id(0)
    start = start_ref[0]
    lane = jax.lax.broadcasted_iota(jnp.int32, (Q, 128), 1)

    q = q_ref[...]
    denom = jnp.clip(jnp.sum(jnp.abs(q), axis=1, keepdims=True), 1e-12, None)
    qn = q / denom

    @pl.when(i == 0)
    def _init():
        ts = jax.lax.dot_general(
            qn, t_ref[...], (((1,), (1,)), ((), ())),
            preferred_element_type=jnp.float32,
        )
        tv, tx = _sort_block(ts, lane, NBLK * BLK, start, TAIL_BLK,
                             masked_tail=True)
        rv_ref[...] = jnp.concatenate(tv, axis=1)
        ri_ref[...] = jnp.concatenate(tx, axis=1)

    base = i * BLK
    groups = []
    for g in range(NGRP):
        s_g = jax.lax.dot_general(
            qn, e_ref[g * 1024:(g + 1) * 1024, :], (((1,), (1,)), ((), ())),
            preferred_element_type=jnp.float32,
        )
        gv, gx = [], []
        for c in range(8):
            off = g * 1024 + c * 128
            x = s_g[:, c * 128:(c + 1) * 128]
            bad = (lane + (base + off - start)).astype(jnp.uint32) < DOC_LEN
            gv.append(jnp.where(bad, _NEG_INF, x))
            gx.append(jnp.full((Q, 128), base + off, jnp.int32))
        _sort8(gv, gx)
        groups.append((gv, gx))
    while len(groups) > 1:
        groups = [
            _top8_merge(groups[k][0], groups[k][1],
                        groups[k + 1][0], groups[k + 1][1])
            for k in range(0, len(groups), 2)
        ]
    nv, nx = groups[0]

    cv = [rv_ref[:, j * 128:(j + 1) * 128] for j in range(TOPK)]
    cx = [ri_ref[:, j * 128:(j + 1) * 128] for j in range(TOPK)]
    mv, mx = _top8_merge(cv, cx, nv, nx)
    rv_ref[...] = jnp.concatenate(mv, axis=1)
    ri_ref[...] = jnp.concatenate(mx, axis=1)

    @pl.when(i == NBLK - 1)
    def _emit():
        fv = list(mv)
        fx = [mx[j] + lane for j in range(TOPK)]
        ov, oi = [], []
        for _ in range(TOPK):
            m = jnp.max(fv[0], axis=1, keepdims=True)
            am = jnp.min(jnp.where(fv[0] == m, fx[0], _BIG_I32),
                         axis=1, keepdims=True)
            ov.append(m)
            oi.append(am)
            hit = (fv[0] == m) & (fx[0] == am)
            for j in range(TOPK - 1):
                fv[j] = jnp.where(hit, fv[j + 1], fv[j])
                fx[j] = jnp.where(hit, fx[j + 1], fx[j])
            fv[TOPK - 1] = jnp.where(hit, _NEG_INF, fv[TOPK - 1])
        vals_out[...] = jnp.concatenate(ov, axis=1).T
        idx_out[...] = jnp.concatenate(oi, axis=1).T


def kernel(query, embeddings, doc_id):
    corpus = embeddings.T
    start = (jnp.asarray(doc_id, jnp.int32) * DOC_LEN).reshape((1,))
    grid_spec = pltpu.PrefetchScalarGridSpec(
        num_scalar_prefetch=1,
        grid=(NBLK,),
        in_specs=[
            pl.BlockSpec((Q, D), lambda i, s: (0, 0)),
            pl.BlockSpec((BLK, D), lambda i, s: (i, 0)),
            pl.BlockSpec((TAIL_BLK, D), lambda i, s: (TAIL_IDX, 0)),
        ],
        out_specs=[
            pl.BlockSpec((TOPK, Q), lambda i, s: (0, 0)),
            pl.BlockSpec((TOPK, Q), lambda i, s: (0, 0)),
        ],
        scratch_shapes=[
            pltpu.VMEM((Q, TOPK * 128), jnp.float32),
            pltpu.VMEM((Q, TOPK * 128), jnp.int32),
        ],
    )
    values, indices = pl.pallas_call(
        _topk_kernel,
        grid_spec=grid_spec,
        out_shape=[
            jax.ShapeDtypeStruct((TOPK, Q), jnp.float32),
            jax.ShapeDtypeStruct((TOPK, Q), jnp.int32),
        ],
    )(start, query, corpus, corpus)
    return values.T, indices.T


# speedup vs baseline: 1.1836x; 1.0042x over previous
import jax
import jax.numpy as jnp
from jax.experimental import pallas as pl
from jax.experimental.pallas import tpu as pltpu

TOPK = 8
DOC_LEN = 100
N_KEYS = 100000
D = 128
Q = 64

BLK = 16384
SUB = BLK // 128
NGRP = SUB // 8
NBLK = N_KEYS // BLK
TAIL_BLK = 2048
TAIL_IDX = N_KEYS // BLK * (BLK // TAIL_BLK)

_NEG_INF = float("-inf")
_BIG_I32 = 2**30

_SORT8 = [
    (0, 1), (2, 3), (4, 5), (6, 7),
    (0, 2), (1, 3), (4, 6), (5, 7),
    (1, 2), (5, 6),
    (0, 4), (1, 5), (2, 6), (3, 7),
    (2, 4), (3, 5),
    (1, 2), (3, 4), (5, 6),
]
_BMERGE8 = [
    (0, 4), (1, 5), (2, 6), (3, 7),
    (0, 2), (1, 3), (4, 6), (5, 7),
    (0, 1), (2, 3), (4, 5), (6, 7),
]


def _ce(v, x, a, b):
    c = v[a] >= v[b]
    va, vb = jnp.maximum(v[a], v[b]), jnp.minimum(v[a], v[b])
    xa, xb = jnp.where(c, x[a], x[b]), jnp.where(c, x[b], x[a])
    v[a], v[b], x[a], x[b] = va, vb, xa, xb


def _sort8(v, x):
    for a, b in _SORT8:
        _ce(v, x, a, b)


def _top8_merge(av, ax, bv, bx):
    mv, mx = [], []
    for j in range(TOPK):
        c = av[j] >= bv[TOPK - 1 - j]
        mv.append(jnp.maximum(av[j], bv[TOPK - 1 - j]))
        mx.append(jnp.where(c, ax[j], bx[TOPK - 1 - j]))
    for a, b in _BMERGE8:
        _ce(mv, mx, a, b)
    return mv, mx


def _sort_block(s, lane, base, start, width, masked_tail):
    groups = []
    for g in range(width // 1024):
        gv, gx = [], []
        for c in range(8):
            off = g * 1024 + c * 128
            x = s[:, off:off + 128]
            bad = (lane + (base + off - start)).astype(jnp.uint32) < DOC_LEN
            if masked_tail:
                bad = bad | (lane >= (N_KEYS - base - off))
            gv.append(jnp.where(bad, _NEG_INF, x))
            gx.append(jnp.full((Q, 128), base + off, jnp.int32))
        _sort8(gv, gx)
        groups.append((gv, gx))
    while len(groups) > 1:
        groups = [
            _top8_merge(groups[k][0], groups[k][1],
                        groups[k + 1][0], groups[k + 1][1])
            for k in range(0, len(groups), 2)
        ]
    return groups[0]


def _topk_kernel(start_ref, q_ref, e_ref, t_ref, vals_out, idx_out,
                 rv_ref, ri_ref):
    i = pl.program_id(0)
    start = start_ref[0]
    lane = jax.lax.broadcasted_iota(jnp.int32, (Q, 128), 1)

    q = q_ref[...]
    denom = jnp.clip(jnp.sum(jnp.abs(q), axis=1, keepdims=True), 1e-12, None)
    qn = q / denom

    @pl.when(i == 0)
    def _init():
        ts = jax.lax.dot_general(
            qn, t_ref[...], (((1,), (1,)), ((), ())),
            preferred_element_type=jnp.float32,
        )
        tv, tx = _sort_block(ts, lane, NBLK * BLK, start, TAIL_BLK,
                             masked_tail=True)
        rv_ref[...] = jnp.concatenate(tv, axis=1)
        ri_ref[...] = jnp.concatenate(tx, axis=1)

    base = i * BLK
    groups = []
    for g in range(NGRP):
        s_g = jax.lax.dot_general(
            qn, e_ref[g * 1024:(g + 1) * 1024, :], (((1,), (1,)), ((), ())),
            preferred_element_type=jnp.float32,
        )
        gv, gx = [], []
        for c in range(8):
            off = g * 1024 + c * 128
            x = s_g[:, c * 128:(c + 1) * 128]
            bad = (lane + (base + off - start)).astype(jnp.uint32) < DOC_LEN
            gv.append(jnp.where(bad, _NEG_INF, x))
            gx.append(jnp.full((Q, 128), base + off, jnp.int32))
        _sort8(gv, gx)
        groups.append((gv, gx))
    while len(groups) > 1:
        groups = [
            _top8_merge(groups[k][0], groups[k][1],
                        groups[k + 1][0], groups[k + 1][1])
            for k in range(0, len(groups), 2)
        ]
    nv, nx = groups[0]

    cv = [rv_ref[:, j * 128:(j + 1) * 128] for j in range(TOPK)]
    cx = [ri_ref[:, j * 128:(j + 1) * 128] for j in range(TOPK)]
    mv, mx = _top8_merge(cv, cx, nv, nx)
    rv_ref[...] = jnp.concatenate(mv, axis=1)
    ri_ref[...] = jnp.concatenate(mx, axis=1)

    @pl.when(i == NBLK - 1)
    def _emit():
        fv = list(mv)
        fx = [mx[j] + lane for j in range(TOPK)]
        ov, oi = [], []
        for _ in range(TOPK):
            m = jnp.max(fv[0], axis=1, keepdims=True)
            am = jnp.min(jnp.where(fv[0] == m, fx[0], _BIG_I32),
                         axis=1, keepdims=True)
            ov.append(m)
            oi.append(am)
            hit = (fv[0] == m) & (fx[0] == am)
            for j in range(TOPK - 1):
                fv[j] = jnp.where(hit, fv[j + 1], fv[j])
                fx[j] = jnp.where(hit, fx[j + 1], fx[j])
            fv[TOPK - 1] = jnp.where(hit, _NEG_INF, fv[TOPK - 1])
        vals_out[...] = jnp.concatenate(ov, axis=1).T
        idx_out[...] = jnp.concatenate(oi, axis=1).T


def kernel(query, embeddings, doc_id):
    corpus = embeddings.T
    start = (jnp.asarray(doc_id, jnp.int32) * DOC_LEN).reshape((1,))
    grid_spec = pltpu.PrefetchScalarGridSpec(
        num_scalar_prefetch=1,
        grid=(NBLK,),
        in_specs=[
            pl.BlockSpec((Q, D), lambda i, s: (0, 0)),
            pl.BlockSpec((BLK, D), lambda i, s: (i, 0)),
            pl.BlockSpec((TAIL_BLK, D), lambda i, s: (TAIL_IDX, 0)),
        ],
        out_specs=[
            pl.BlockSpec((TOPK, Q), lambda i, s: (0, 0)),
            pl.BlockSpec((TOPK, Q), lambda i, s: (0, 0)),
        ],
        scratch_shapes=[
            pltpu.VMEM((Q, TOPK * 128), jnp.float32),
            pltpu.VMEM((Q, TOPK * 128), jnp.int32),
        ],
    )
    values, indices = pl.pallas_call(
        _topk_kernel,
        grid_spec=grid_spec,
        out_shape=[
            jax.ShapeDtypeStruct((TOPK, Q), jnp.float32),
            jax.ShapeDtypeStruct((TOPK, Q), jnp.int32),
        ],
    )(start, query, corpus, corpus)
    return values.T, indices.T
